# baseline (speedup 1.0000x reference)
import functools

import jax
import jax.numpy as jnp
from jax import lax
from jax.experimental import pallas as pl
from jax.experimental.pallas import tpu as pltpu
from jax.experimental.pallas import tpu_sc as plsc

LANES = 16
NW = 32
LP = 56
NBUF = 8


def _sc_pool_body(L, RPW, idx_hbm, len_hbm, table_hbm, out_hbm,
                  idx_v, len_v, out_v, bufs, sems):
    D = table_hbm.shape[1]
    DK = D // LANES
    wid = lax.axis_index("s") * 2 + lax.axis_index("c")
    base = wid * RPW

    pltpu.sync_copy(idx_hbm.at[pl.ds(base * LP, RPW * LP)], idx_v)
    pltpu.sync_copy(len_hbm.at[pl.ds(base, RPW)], len_v.at[pl.ds(0, RPW)])

    def _nb(r):
        m_r = len_v[pl.ds(r, LANES)][0]
        return jnp.minimum(jnp.maximum(m_r, 1), L)

    def _each_size(r, slot, fn):
        nb = _nb(r)
        k = (nb + 7) // 8
        for kk in range(1, LP // 8 + 1):
            @pl.when(k == kk)
            def _():
                fn(pltpu.make_async_copy(
                    table_hbm.at[idx_v.at[pl.ds(r * LP, 8 * kk)]],
                    bufs[slot].at[pl.ds(0, 8 * kk), :], sems[slot]))

    def _start(r, slot):
        _each_size(r, slot, lambda cp: cp.start())

    def _wait(r, slot):
        _each_size(r, slot, lambda cp: cp.wait())

    def _accum(r, slot):
        buf = bufs[slot]
        nb = _nb(r)
        init = tuple(jnp.zeros((LANES,), jnp.float32) for _ in range(DK))

        @pl.loop(0, nb, init_carry=init)
        def accs(j, acc):
            return tuple(
                acc[k] + buf[j, pl.ds(k * LANES, LANES)]
                for k in range(DK))

        for k in range(DK):
            out_v[r, pl.ds(k * LANES, LANES)] = accs[k]

    for slot in range(NBUF):
        _start(slot, slot)

    def outer(r0, _):
        for slot in range(NBUF):
            r = r0 + slot
            _wait(r, slot)
            _accum(r, slot)

            @pl.when(r + NBUF < RPW)
            def _():
                _start(r + NBUF, slot)
        return 0

    lax.fori_loop(0, RPW // NBUF, lambda i, c: outer(i * NBUF, c), 0)

    pltpu.sync_copy(out_v, out_hbm.at[pl.ds(base, RPW), :])


def _sc_pool(idx_flat, seq_lengths, table, L):
    B = seq_lengths.shape[0]
    D = table.shape[1]
    RPW = B // NW
    mesh = plsc.VectorSubcoreMesh(core_axis_name="c", subcore_axis_name="s")
    f = pl.kernel(
        functools.partial(_sc_pool_body, L, RPW),
        out_type=jax.ShapeDtypeStruct((B, D), jnp.float32),
        mesh=mesh,
        scratch_types=dict(
            idx_v=pltpu.VMEM((RPW * LP,), jnp.int32),
            len_v=pltpu.VMEM((RPW + LANES,), jnp.int32),
            out_v=pltpu.VMEM((RPW, D), jnp.float32),
            bufs=[pltpu.VMEM((LP, D), jnp.float32) for _ in range(NBUF)],
            sems=[pltpu.SemaphoreType.DMA for _ in range(NBUF)],
        ),
    )
    return f(idx_flat, seq_lengths, table)


def _mm_body(s_ref, m_ref, w_ref, b_ref, o_ref):
    m = jnp.maximum(m_ref[...].astype(jnp.float32), 1.0)
    o_ref[...] = jnp.dot(s_ref[...] / m, w_ref[...],
                         preferred_element_type=jnp.float32) + b_ref[...]


def _tc_transform(sums, seq_lengths, W, b):
    B, D = sums.shape
    E = W.shape[1]
    BM = 4096
    return pl.pallas_call(
        _mm_body,
        grid=(B // BM,),
        in_specs=[
            pl.BlockSpec((BM, D), lambda i: (i, 0)),
            pl.BlockSpec((BM, 1), lambda i: (i, 0)),
            pl.BlockSpec((D, E), lambda i: (0, 0)),
            pl.BlockSpec((1, E), lambda i: (0, 0)),
        ],
        out_specs=pl.BlockSpec((BM, E), lambda i: (i, 0)),
        out_shape=jax.ShapeDtypeStruct((B, E), jnp.float32),
    )(sums, seq_lengths.reshape(B, 1), W, b.reshape(1, E))


def kernel(token_indices, seq_lengths, table, W, b):
    B, L = token_indices.shape
    V = table.shape[0]
    filler = ((jnp.arange(B, dtype=jnp.int32)[:, None] * (LP - L)
               + jnp.arange(LP - L, dtype=jnp.int32)[None, :]) * 97) % V
    idx_p = jnp.concatenate([token_indices, filler], axis=1)
    sums = _sc_pool(idx_p.reshape(-1), seq_lengths, table, L)
    return _tc_transform(sums, seq_lengths, W, b)


# speedup vs baseline: 1.0109x; 1.0109x over previous
import functools

import jax
import jax.numpy as jnp
from jax import lax
from jax.experimental import pallas as pl
from jax.experimental.pallas import tpu as pltpu
from jax.experimental.pallas import tpu_sc as plsc

LANES = 16
NW = 32
LP = 56
NBUF = 8


def _sc_pool_body(L, RPW, idx_hbm, len_hbm, table_hbm, out_hbm,
                  idx_v, len_v, out_v, bufs, sems):
    D = table_hbm.shape[1]
    DK = D // LANES
    wid = lax.axis_index("s") * 2 + lax.axis_index("c")
    base = wid * RPW

    pltpu.sync_copy(idx_hbm.at[pl.ds(base * LP, RPW * LP)], idx_v)
    pltpu.sync_copy(len_hbm.at[pl.ds(base, RPW)], len_v.at[pl.ds(0, RPW)])

    def _nb(r):
        m_r = len_v[pl.ds(r, LANES)][0]
        return jnp.minimum(jnp.maximum(m_r, 1), L)

    def _each_size(r, slot, fn):
        nb = _nb(r)
        k = (nb + 7) // 8
        for kk in range(1, LP // 8 + 1):
            @pl.when(k == kk)
            def _():
                fn(pltpu.make_async_copy(
                    table_hbm.at[idx_v.at[pl.ds(r * LP, 8 * kk)]],
                    bufs[slot].at[pl.ds(0, 8 * kk), :], sems[slot]))

    def _start(r, slot):
        _each_size(r, slot, lambda cp: cp.start())

    def _wait(r, slot):
        _each_size(r, slot, lambda cp: cp.wait())

    def _accum(r, slot):
        buf = bufs[slot]
        nb = _nb(r)
        init = tuple(jnp.zeros((LANES,), jnp.float32) for _ in range(DK))

        @pl.loop(0, nb, init_carry=init)
        def accs(j, acc):
            return tuple(
                acc[k] + buf[j, pl.ds(k * LANES, LANES)]
                for k in range(DK))

        for k in range(DK):
            out_v[r, pl.ds(k * LANES, LANES)] = accs[k]

    for slot in range(NBUF):
        _start(slot, slot)

    def outer(r0, _):
        for slot in range(NBUF):
            r = r0 + slot
            _wait(r, slot)
            _accum(r, slot)

            @pl.when(r + NBUF < RPW)
            def _():
                _start(r + NBUF, slot)
        return 0

    lax.fori_loop(0, RPW // NBUF, lambda i, c: outer(i * NBUF, c), 0)

    pltpu.sync_copy(out_v, out_hbm.at[pl.ds(base, RPW), :])


def _sc_pool(idx_flat, seq_lengths, table, L):
    B = seq_lengths.shape[0]
    D = table.shape[1]
    RPW = B // NW
    mesh = plsc.VectorSubcoreMesh(core_axis_name="c", subcore_axis_name="s")
    f = pl.kernel(
        functools.partial(_sc_pool_body, L, RPW),
        out_type=jax.ShapeDtypeStruct((B, D), jnp.float32),
        mesh=mesh,
        scratch_types=dict(
            idx_v=pltpu.VMEM((RPW * LP,), jnp.int32),
            len_v=pltpu.VMEM((RPW + LANES,), jnp.int32),
            out_v=pltpu.VMEM((RPW, D), jnp.float32),
            bufs=[pltpu.VMEM((LP, D), jnp.float32) for _ in range(NBUF)],
            sems=[pltpu.SemaphoreType.DMA for _ in range(NBUF)],
        ),
    )
    return f(idx_flat, seq_lengths, table)


def _mm_body(s_ref, m_ref, w_ref, b_ref, o_ref):
    m = jnp.maximum(m_ref[...].astype(jnp.float32), 1.0)
    o_ref[...] = jnp.dot(s_ref[...] / m, w_ref[...],
                         preferred_element_type=jnp.float32) + b_ref[...]


def _tc_transform(sums, seq_lengths, W, b):
    B, D = sums.shape
    E = W.shape[1]
    BM = 2048
    return pl.pallas_call(
        _mm_body,
        grid=(B // BM,),
        in_specs=[
            pl.BlockSpec((BM, D), lambda i: (i, 0)),
            pl.BlockSpec((BM, 1), lambda i: (i, 0)),
            pl.BlockSpec((D, E), lambda i: (0, 0)),
            pl.BlockSpec((1, E), lambda i: (0, 0)),
        ],
        out_specs=pl.BlockSpec((BM, E), lambda i: (i, 0)),
        out_shape=jax.ShapeDtypeStruct((B, E), jnp.float32),
    )(sums, seq_lengths.reshape(B, 1), W, b.reshape(1, E))


def kernel(token_indices, seq_lengths, table, W, b):
    B, L = token_indices.shape
    V = table.shape[0]
    filler = ((jnp.arange(B, dtype=jnp.int32)[:, None] * (LP - L)
               + jnp.arange(LP - L, dtype=jnp.int32)[None, :]) * 97) % V
    idx_p = jnp.concatenate([token_indices, filler], axis=1)
    sums = _sc_pool(idx_p.reshape(-1), seq_lengths, table, L)
    return _tc_transform(sums, seq_lengths, W, b)
